# baseline (speedup 1.0000x reference)
import functools

import jax
import jax.numpy as jnp
from jax import lax
from jax.experimental import pallas as pl
from jax.experimental.pallas import tpu as pltpu
from jax.experimental.pallas import tpu_sc as plsc

B, L, D = 4, 4096, 768
MAX_POS = 1024
NC, NS, LANES = 2, 16, 16
NW = NC * NS
TOK = B * L
TPW = TOK // NW
C = 32
NCH = TPW // C
DP = D // 2
DV = DP // LANES


def _sc_body(c0, c1, c2, c3, w, h, tlx, tly, brx, bry, wemb, hemb, out,
             idx_v, stage, whp, wv, hv, isem, gsem0, gsem1, ssem0, ssem1):
    wid = lax.axis_index("s") * NC + lax.axis_index("c")
    base = pl.multiple_of(wid * TPW, TPW)
    b = wid // (NW // B)

    stage_bf = stage.bitcast(jnp.bfloat16)
    whp_bf = whp.bitcast(jnp.bfloat16)

    descs = [pltpu.async_copy(c.at[pl.ds(base, TPW)], idx_v.at[t], isem)
             for t, c in enumerate((c0, c1, c2, c3))]
    for d in descs:
        d.wait()
    for t in range(4):
        for j in range(TPW // LANES):
            sl = pl.ds(j * LANES, LANES)
            idx_v[t, sl] = jnp.minimum(idx_v[t, sl], MAX_POS - 1)

    bvec = jnp.full((LANES,), b, jnp.int32)
    pltpu.async_copy(w.at[bvec], wv, isem).wait()
    pltpu.async_copy(h.at[bvec], hv, isem).wait()
    wv[...] = jnp.minimum(wv[...], MAX_POS - 1)
    hv[...] = jnp.minimum(hv[...], MAX_POS - 1)
    pltpu.async_copy(wemb.at[wv], stage.at[0, 0, pl.ds(0, LANES)], isem).wait()
    pltpu.async_copy(hemb.at[hv], stage.at[0, 1, pl.ds(0, LANES)], isem).wait()
    pr = pl.ds(0, 2)
    for s in range(DV):
        sl = pl.ds(s * LANES, LANES)
        whp_bf[pr, sl] = stage_bf[0, 0, pr, sl] + stage_bf[0, 1, pr, sl]

    def gather_descs(k, p, sem):
        ksl = pl.ds(pl.multiple_of(k * C, C), C)
        return [pltpu.make_async_copy(tab.at[idx_v.at[t, ksl]],
                                      stage.at[p, t], sem)
                for t, tab in enumerate((tlx, tly, brx, bry))]

    def issue_gathers(k, p, sem):
        for d in gather_descs(k, p, sem):
            d.start()

    def wait_gathers(k, p, sem):
        for d in gather_descs(k, p, sem):
            d.wait()

    def scatter_desc(k, p, sem):
        koff = pl.multiple_of(k * C, C)
        return pltpu.make_async_copy(stage.at[p, 0],
                                     out.at[pl.ds(base + koff, C)], sem)

    def combine(p):
        def body(i, carry):
            rs = pl.ds(pl.multiple_of(2 * i, 2), 2)
            for s in range(DV):
                sl = pl.ds(s * LANES, LANES)
                a = stage_bf[p, 0, rs, sl] + stage_bf[p, 1, rs, sl]
                bb = stage_bf[p, 2, rs, sl] + stage_bf[p, 3, rs, sl]
                stage_bf[p, 0, rs, sl] = a + bb + whp_bf[pr, sl]
            return carry
        lax.fori_loop(0, C, body, 0)

    issue_gathers(0, 0, gsem0)

    def pair(j, carry):
        k0 = pl.multiple_of(2 * j, 2)
        k1 = k0 + 1
        wait_gathers(k0, 0, gsem0)

        @pl.when(j > 0)
        def _():
            scatter_desc(k0, 1, ssem1).wait()
        issue_gathers(k1, 1, gsem1)
        combine(0)
        scatter_desc(k0, 0, ssem0).start()
        wait_gathers(k1, 1, gsem1)
        scatter_desc(k0, 0, ssem0).wait()

        @pl.when(k1 + 1 < NCH)
        def _():
            issue_gathers(k1 + 1, 0, gsem0)
        combine(1)
        scatter_desc(k1, 1, ssem1).start()
        return carry
    lax.fori_loop(0, NCH // 2, pair, 0)
    scatter_desc(NCH - 1, 1, ssem1).wait()


_mesh = plsc.VectorSubcoreMesh(core_axis_name="c", subcore_axis_name="s")

_spatial_sum = functools.partial(
    pl.kernel,
    out_type=jax.ShapeDtypeStruct((TOK, DP), jnp.int32),
    mesh=_mesh,
    scratch_types=[
        pltpu.VMEM((4, TPW), jnp.int32),
        pltpu.VMEM((2, 4, C, DP), jnp.int32),
        pltpu.VMEM((1, DP), jnp.int32),
        pltpu.VMEM((LANES,), jnp.int32),
        pltpu.VMEM((LANES,), jnp.int32),
        pltpu.SemaphoreType.DMA,
        pltpu.SemaphoreType.DMA,
        pltpu.SemaphoreType.DMA,
        pltpu.SemaphoreType.DMA,
        pltpu.SemaphoreType.DMA,
    ],
)(_sc_body)


def _pack(tab):
    tab16 = tab.astype(jnp.bfloat16).reshape(MAX_POS, DP, 2)
    return lax.bitcast_convert_type(tab16, jnp.int32)


def kernel(coordinates, width, height, top_left_x, top_left_y,
           bottom_right_x, bottom_right_y, width_emb, height_emb):
    cidx = coordinates.reshape(TOK, 4).astype(jnp.int32)
    c0, c1, c2, c3 = (cidx[:, t] for t in range(4))
    out = _spatial_sum(c0, c1, c2, c3,
                       width.astype(jnp.int32), height.astype(jnp.int32),
                       _pack(top_left_x), _pack(top_left_y),
                       _pack(bottom_right_x), _pack(bottom_right_y),
                       _pack(width_emb), _pack(height_emb))
    out16 = lax.bitcast_convert_type(out, jnp.bfloat16).reshape(TOK, D)
    return out16.astype(jnp.float32).reshape(B, L, D)


# speedup vs baseline: 2.5000x; 2.5000x over previous
import functools

import jax
import jax.numpy as jnp
from jax import lax
from jax.experimental import pallas as pl
from jax.experimental.pallas import tpu as pltpu
from jax.experimental.pallas import tpu_sc as plsc

B, L, D = 4, 4096, 768
MAX_POS = 1024
NC, NS, LANES = 2, 16, 16
NW = NC * NS
TOK = B * L
TPW = TOK // NW
C = 32
NCH = TPW // C
DP = D // 2
DV = DP // LANES


def _sc_body(c0, c1, c2, c3, w, h, tlx, tly, brx, bry, wemb, hemb, out,
             idx_v, stage, whp, wv, hv, isem, gsem0, gsem1, ssem0, ssem1):
    wid = lax.axis_index("s") * NC + lax.axis_index("c")
    base = pl.multiple_of(wid * TPW, TPW)
    b = wid // (NW // B)

    stage_bf = stage.bitcast(jnp.bfloat16)
    whp_bf = whp.bitcast(jnp.bfloat16)

    descs = [pltpu.async_copy(c.at[pl.ds(base, TPW)], idx_v.at[t], isem)
             for t, c in enumerate((c0, c1, c2, c3))]
    for d in descs:
        d.wait()
    for t in range(4):
        for j in range(TPW // LANES):
            sl = pl.ds(j * LANES, LANES)
            idx_v[t, sl] = jnp.minimum(idx_v[t, sl], MAX_POS - 1)

    bvec = jnp.full((LANES,), b, jnp.int32)
    pltpu.async_copy(w.at[bvec], wv, isem).wait()
    pltpu.async_copy(h.at[bvec], hv, isem).wait()
    wv[...] = jnp.minimum(wv[...], MAX_POS - 1)
    hv[...] = jnp.minimum(hv[...], MAX_POS - 1)
    pltpu.async_copy(wemb.at[wv], stage.at[0, 0, pl.ds(0, LANES)], isem).wait()
    pltpu.async_copy(hemb.at[hv], stage.at[0, 1, pl.ds(0, LANES)], isem).wait()
    pr = pl.ds(0, 2)
    for s in range(DV):
        sl = pl.ds(s * LANES, LANES)
        whp_bf[pr, sl] = stage_bf[0, 0, pr, sl] + stage_bf[0, 1, pr, sl]

    def gather_descs(k, p, sem):
        ksl = pl.ds(pl.multiple_of(k * C, C), C)
        return [pltpu.make_async_copy(tab.at[idx_v.at[t, ksl]],
                                      stage.at[p, t], sem)
                for t, tab in enumerate((tlx, tly, brx, bry))]

    def issue_gathers(k, p, sem):
        for d in gather_descs(k, p, sem):
            d.start()

    def wait_gathers(k, p, sem):
        for d in gather_descs(k, p, sem):
            d.wait()

    def scatter_desc(k, p, sem):
        koff = pl.multiple_of(k * C, C)
        return pltpu.make_async_copy(stage.at[p, 0],
                                     out.at[pl.ds(base + koff, C)], sem)

    def combine(p):
        def body(i, carry):
            rs = pl.ds(pl.multiple_of(2 * i, 2), 2)
            for s in range(DV):
                sl = pl.ds(s * LANES, LANES)
                a = stage_bf[p, 0, rs, sl] + stage_bf[p, 1, rs, sl]
                bb = stage_bf[p, 2, rs, sl] + stage_bf[p, 3, rs, sl]
                stage_bf[p, 0, rs, sl] = a + bb + whp_bf[pr, sl]
            return carry
        lax.fori_loop(0, C, body, 0)

    issue_gathers(0, 0, gsem0)

    def pair(j, carry):
        k0 = pl.multiple_of(2 * j, 2)
        k1 = k0 + 1
        wait_gathers(k0, 0, gsem0)

        @pl.when(j > 0)
        def _():
            scatter_desc(k0, 1, ssem1).wait()
        issue_gathers(k1, 1, gsem1)
        combine(0)
        scatter_desc(k0, 0, ssem0).start()
        wait_gathers(k1, 1, gsem1)
        scatter_desc(k0, 0, ssem0).wait()

        @pl.when(k1 + 1 < NCH)
        def _():
            issue_gathers(k1 + 1, 0, gsem0)
        combine(1)
        scatter_desc(k1, 1, ssem1).start()
        return carry
    lax.fori_loop(0, NCH // 2, pair, 0)
    scatter_desc(NCH - 1, 1, ssem1).wait()


_mesh = plsc.VectorSubcoreMesh(core_axis_name="c", subcore_axis_name="s")

_spatial_sum = functools.partial(
    pl.kernel,
    out_type=jax.ShapeDtypeStruct((TOK, DP), jnp.int32),
    mesh=_mesh,
    scratch_types=[
        pltpu.VMEM((4, TPW), jnp.int32),
        pltpu.VMEM((2, 4, C, DP), jnp.int32),
        pltpu.VMEM((1, DP), jnp.int32),
        pltpu.VMEM((LANES,), jnp.int32),
        pltpu.VMEM((LANES,), jnp.int32),
        pltpu.SemaphoreType.DMA,
        pltpu.SemaphoreType.DMA,
        pltpu.SemaphoreType.DMA,
        pltpu.SemaphoreType.DMA,
        pltpu.SemaphoreType.DMA,
    ],
)(_sc_body)


def _tc_pack_body(*refs):
    for i_ref, o_ref in zip(refs[:6], refs[6:]):
        xb = i_ref[...].astype(jnp.bfloat16)
        u = lax.bitcast_convert_type(xb, jnp.uint16).astype(jnp.int32)
        lo, hi = u[:, :DP], u[:, DP:]
        o_ref[...] = lo | (hi << 16)


_tc_pack = pl.pallas_call(
    _tc_pack_body,
    out_shape=[jax.ShapeDtypeStruct((MAX_POS, DP), jnp.int32)] * 6,
)

_RB = 1024


def _tc_unpack_body(i_ref, o_ref):
    w = i_ref[...]
    lo = lax.bitcast_convert_type((w & 0xFFFF).astype(jnp.uint16),
                                  jnp.bfloat16)
    hi = lax.bitcast_convert_type(
        lax.shift_right_logical(w, 16).astype(jnp.uint16), jnp.bfloat16)
    o_ref[:, :DP] = lo.astype(jnp.float32)
    o_ref[:, DP:] = hi.astype(jnp.float32)


_tc_unpack = pl.pallas_call(
    _tc_unpack_body,
    grid=(TOK // _RB,),
    in_specs=[pl.BlockSpec((_RB, DP), lambda i: (i, 0))],
    out_specs=pl.BlockSpec((_RB, D), lambda i: (i, 0)),
    out_shape=jax.ShapeDtypeStruct((TOK, D), jnp.float32),
)


def kernel(coordinates, width, height, top_left_x, top_left_y,
           bottom_right_x, bottom_right_y, width_emb, height_emb):
    cidx = coordinates.reshape(TOK, 4).astype(jnp.int32)
    c0, c1, c2, c3 = (cidx[:, t] for t in range(4))
    packed = _tc_pack(top_left_x, top_left_y, bottom_right_x, bottom_right_y,
                      width_emb, height_emb)
    out = _spatial_sum(c0, c1, c2, c3,
                       width.astype(jnp.int32), height.astype(jnp.int32),
                       *packed)
    return _tc_unpack(out).reshape(B, L, D)


# speedup vs baseline: 2.9307x; 1.1723x over previous
import functools

import jax
import jax.numpy as jnp
from jax import lax
from jax.experimental import pallas as pl
from jax.experimental.pallas import tpu as pltpu
from jax.experimental.pallas import tpu_sc as plsc

B, L, D = 4, 4096, 768
MAX_POS = 1024
NC, NS, LANES = 2, 16, 16
NW = NC * NS
TOK = B * L
TPW = TOK // NW
C = 16
NCH = TPW // C
DP = D // 2
DV = DP // LANES


def _sc_body(c0, c1, c2, c3, w, h, tlx, tly, brx, bry, wemb, hemb, out,
             idx_v, stage, obuf, whp, wv, hv,
             isem, gsem0, gsem1, ssem0, ssem1):
    wid = lax.axis_index("s") * NC + lax.axis_index("c")
    base = pl.multiple_of(wid * TPW, TPW)
    b = wid // (NW // B)

    stage_bf = stage.bitcast(jnp.bfloat16)
    whp_bf = whp.bitcast(jnp.bfloat16)

    descs = [pltpu.async_copy(c.at[pl.ds(base, TPW)], idx_v.at[t], isem)
             for t, c in enumerate((c0, c1, c2, c3))]
    for d in descs:
        d.wait()
    for t in range(4):
        for j in range(TPW // LANES):
            sl = pl.ds(j * LANES, LANES)
            idx_v[t, sl] = jnp.minimum(idx_v[t, sl], MAX_POS - 1)

    bvec = jnp.full((LANES,), b, jnp.int32)
    pltpu.async_copy(w.at[bvec], wv, isem).wait()
    pltpu.async_copy(h.at[bvec], hv, isem).wait()
    wv[...] = jnp.minimum(wv[...], MAX_POS - 1)
    hv[...] = jnp.minimum(hv[...], MAX_POS - 1)
    pltpu.async_copy(wemb.at[wv], stage.at[0, 0, pl.ds(0, LANES)], isem).wait()
    pltpu.async_copy(hemb.at[hv], stage.at[0, 1, pl.ds(0, LANES)], isem).wait()
    pr = pl.ds(0, 2)
    for s in range(DV):
        sl = pl.ds(s * LANES, LANES)
        whp_bf[pr, sl] = stage_bf[0, 0, pr, sl] + stage_bf[0, 1, pr, sl]
    whvecs = [whp_bf[pr, pl.ds(s * LANES, LANES)] for s in range(DV)]

    def gather_descs(k, p, sem):
        ksl = pl.ds(pl.multiple_of(k * C, C), C)
        return [pltpu.make_async_copy(tab.at[idx_v.at[t, ksl]],
                                      stage.at[p, t], sem)
                for t, tab in enumerate((tlx, tly, brx, bry))]

    def issue_gathers(k, p, sem):
        for d in gather_descs(k, p, sem):
            d.start()

    def wait_gathers(k, p, sem):
        for d in gather_descs(k, p, sem):
            d.wait()

    def scatter_desc(k, p, sem):
        koff = pl.multiple_of(k * C, C)
        return pltpu.make_async_copy(obuf.at[p],
                                     out.at[pl.ds(base + koff, C)], sem)

    def combine(p):
        def body(i, carry):
            rs = pl.ds(pl.multiple_of(2 * i, 2), 2)
            for s in range(DV):
                sl = pl.ds(s * LANES, LANES)
                a = stage_bf[p, 0, rs, sl] + stage_bf[p, 1, rs, sl]
                bb = stage_bf[p, 2, rs, sl] + stage_bf[p, 3, rs, sl]
                stage_bf[p, 0, rs, sl] = a + bb + whvecs[s]
            return carry
        lax.fori_loop(0, C, body, 0)

    def unpack(p):
        def body(i, carry):
            for s in range(DV):
                sl = pl.ds(s * LANES, LANES)
                wrd = stage[p, 0, i, sl]
                obuf[p, i, sl] = lax.bitcast_convert_type(
                    wrd << 16, jnp.float32)
                obuf[p, i, pl.ds(DP + s * LANES, LANES)] = (
                    lax.bitcast_convert_type(wrd & jnp.int32(-65536),
                                             jnp.float32))
            return carry
        lax.fori_loop(0, C, body, 0)

    issue_gathers(0, 0, gsem0)

    def pair(j, carry):
        k0 = pl.multiple_of(2 * j, 2)
        k1 = k0 + 1
        wait_gathers(k0, 0, gsem0)
        issue_gathers(k1, 1, gsem1)
        combine(0)

        @pl.when(j > 0)
        def _():
            scatter_desc(k0, 0, ssem0).wait()
        unpack(0)
        scatter_desc(k0, 0, ssem0).start()
        wait_gathers(k1, 1, gsem1)

        @pl.when(k1 + 1 < NCH)
        def _():
            issue_gathers(k1 + 1, 0, gsem0)
        combine(1)

        @pl.when(j > 0)
        def _():
            scatter_desc(k1, 1, ssem1).wait()
        unpack(1)
        scatter_desc(k1, 1, ssem1).start()
        return carry
    lax.fori_loop(0, NCH // 2, pair, 0)
    scatter_desc(NCH - 2, 0, ssem0).wait()
    scatter_desc(NCH - 1, 1, ssem1).wait()


_mesh = plsc.VectorSubcoreMesh(core_axis_name="c", subcore_axis_name="s")

_spatial_sum = functools.partial(
    pl.kernel,
    out_type=jax.ShapeDtypeStruct((TOK, D), jnp.float32),
    mesh=_mesh,
    scratch_types=[
        pltpu.VMEM((4, TPW), jnp.int32),
        pltpu.VMEM((2, 4, C, DP), jnp.int32),
        pltpu.VMEM((2, C, D), jnp.float32),
        pltpu.VMEM((1, DP), jnp.int32),
        pltpu.VMEM((LANES,), jnp.int32),
        pltpu.VMEM((LANES,), jnp.int32),
        pltpu.SemaphoreType.DMA,
        pltpu.SemaphoreType.DMA,
        pltpu.SemaphoreType.DMA,
        pltpu.SemaphoreType.DMA,
        pltpu.SemaphoreType.DMA,
    ],
)(_sc_body)


def _tc_pack_body(*refs):
    for i_ref, o_ref in zip(refs[:6], refs[6:]):
        xb = i_ref[...].astype(jnp.bfloat16)
        u = lax.bitcast_convert_type(xb, jnp.uint16).astype(jnp.int32)
        lo, hi = u[:, :DP], u[:, DP:]
        o_ref[...] = lo | (hi << 16)


_tc_pack = pl.pallas_call(
    _tc_pack_body,
    out_shape=[jax.ShapeDtypeStruct((MAX_POS, DP), jnp.int32)] * 6,
)


def kernel(coordinates, width, height, top_left_x, top_left_y,
           bottom_right_x, bottom_right_y, width_emb, height_emb):
    cidx = coordinates.reshape(TOK, 4).astype(jnp.int32)
    c0, c1, c2, c3 = (cidx[:, t] for t in range(4))
    packed = _tc_pack(top_left_x, top_left_y, bottom_right_x, bottom_right_y,
                      width_emb, height_emb)
    out = _spatial_sum(c0, c1, c2, c3,
                       width.astype(jnp.int32), height.astype(jnp.int32),
                       *packed)
    return out.reshape(B, L, D)


# speedup vs baseline: 3.6951x; 1.2608x over previous
import functools

import jax
import jax.numpy as jnp
from jax import lax
from jax.experimental import pallas as pl
from jax.experimental.pallas import tpu as pltpu
from jax.experimental.pallas import tpu_sc as plsc

B, L, D = 4, 4096, 768
MAX_POS = 1024
NC, NS, LANES = 2, 16, 16
NW = NC * NS
TOK = B * L
TPW = TOK // NW
C = 16
NCH = TPW // C
DP = D // 2
DV = DP // LANES


def _sc_body(c0, c1, c2, c3, w, h, tlx, tly, brx, bry, wemb, hemb, out,
             idx_v, stage, obuf, whp, wv, hv,
             isem, gsem0, gsem1, ssem0, ssem1):
    wid = lax.axis_index("s") * NC + lax.axis_index("c")
    base = pl.multiple_of(wid * TPW, TPW)
    b = wid // (NW // B)

    stage_bf = stage.bitcast(jnp.bfloat16)
    whp_bf = whp.bitcast(jnp.bfloat16)

    descs = [pltpu.async_copy(c.at[pl.ds(base, TPW)], idx_v.at[t], isem)
             for t, c in enumerate((c0, c1, c2, c3))]
    for d in descs:
        d.wait()
    for t in range(4):
        for j in range(TPW // LANES):
            sl = pl.ds(j * LANES, LANES)
            idx_v[t, sl] = jnp.minimum(idx_v[t, sl], MAX_POS - 1)

    bvec = jnp.full((LANES,), b, jnp.int32)
    pltpu.async_copy(w.at[bvec], wv, isem).wait()
    pltpu.async_copy(h.at[bvec], hv, isem).wait()
    wv[...] = jnp.minimum(wv[...], MAX_POS - 1)
    hv[...] = jnp.minimum(hv[...], MAX_POS - 1)
    pltpu.async_copy(wemb.at[wv], stage.at[0, 0, pl.ds(0, LANES)], isem).wait()
    pltpu.async_copy(hemb.at[hv], stage.at[0, 1, pl.ds(0, LANES)], isem).wait()
    pr = pl.ds(0, 2)
    for s in range(DV):
        sl = pl.ds(s * LANES, LANES)
        whp_bf[pr, sl] = stage_bf[0, 0, pr, sl] + stage_bf[0, 1, pr, sl]
    whvecs = [whp_bf[pr, pl.ds(s * LANES, LANES)] for s in range(DV)]

    def gather_descs(k, p, sem):
        ksl = pl.ds(pl.multiple_of(k * C, C), C)
        return [pltpu.make_async_copy(tab.at[idx_v.at[t, ksl]],
                                      stage.at[p, t], sem)
                for t, tab in enumerate((tlx, tly, brx, bry))]

    def issue_gathers(k, p, sem):
        for d in gather_descs(k, p, sem):
            d.start()

    def wait_gathers(k, p, sem):
        for d in gather_descs(k, p, sem):
            d.wait()

    def scatter_desc(k, p, sem):
        koff = pl.multiple_of(k * C, C)
        return pltpu.make_async_copy(obuf.at[p],
                                     out.at[pl.ds(base + koff, C)], sem)

    def process(p):
        @plsc.parallel_loop(0, C, step=1)
        def body(i):
            rs = pl.ds(pl.multiple_of(2 * i, 2), 2)
            for s in range(DV):
                sl = pl.ds(s * LANES, LANES)
                a = stage_bf[p, 0, rs, sl] + stage_bf[p, 1, rs, sl]
                bb = stage_bf[p, 2, rs, sl] + stage_bf[p, 3, rs, sl]
                stage_bf[p, 0, rs, sl] = a + bb + whvecs[s]
            for s in range(DV):
                sl = pl.ds(s * LANES, LANES)
                wrd = stage[p, 0, i, sl]
                obuf[p, i, sl] = lax.bitcast_convert_type(
                    wrd << 16, jnp.float32)
                obuf[p, i, pl.ds(DP + s * LANES, LANES)] = (
                    lax.bitcast_convert_type(wrd & jnp.int32(-65536),
                                             jnp.float32))

    issue_gathers(0, 0, gsem0)

    def pair(j, carry):
        k0 = pl.multiple_of(2 * j, 2)
        k1 = k0 + 1
        wait_gathers(k0, 0, gsem0)
        issue_gathers(k1, 1, gsem1)
        @pl.when(j > 0)
        def _():
            scatter_desc(k0, 0, ssem0).wait()
        process(0)
        scatter_desc(k0, 0, ssem0).start()
        wait_gathers(k1, 1, gsem1)

        @pl.when(k1 + 1 < NCH)
        def _():
            issue_gathers(k1 + 1, 0, gsem0)
        @pl.when(j > 0)
        def _():
            scatter_desc(k1, 1, ssem1).wait()
        process(1)
        scatter_desc(k1, 1, ssem1).start()
        return carry
    lax.fori_loop(0, NCH // 2, pair, 0)
    scatter_desc(NCH - 2, 0, ssem0).wait()
    scatter_desc(NCH - 1, 1, ssem1).wait()


_mesh = plsc.VectorSubcoreMesh(core_axis_name="c", subcore_axis_name="s")

_spatial_sum = functools.partial(
    pl.kernel,
    out_type=jax.ShapeDtypeStruct((TOK, D), jnp.float32),
    mesh=_mesh,
    scratch_types=[
        pltpu.VMEM((4, TPW), jnp.int32),
        pltpu.VMEM((2, 4, C, DP), jnp.int32),
        pltpu.VMEM((2, C, D), jnp.float32),
        pltpu.VMEM((1, DP), jnp.int32),
        pltpu.VMEM((LANES,), jnp.int32),
        pltpu.VMEM((LANES,), jnp.int32),
        pltpu.SemaphoreType.DMA,
        pltpu.SemaphoreType.DMA,
        pltpu.SemaphoreType.DMA,
        pltpu.SemaphoreType.DMA,
        pltpu.SemaphoreType.DMA,
    ],
)(_sc_body)


def _tc_pack_body(*refs):
    for i_ref, o_ref in zip(refs[:6], refs[6:]):
        xb = i_ref[...].astype(jnp.bfloat16)
        u = lax.bitcast_convert_type(xb, jnp.uint16).astype(jnp.int32)
        lo, hi = u[:, :DP], u[:, DP:]
        o_ref[...] = lo | (hi << 16)


_tc_pack = pl.pallas_call(
    _tc_pack_body,
    out_shape=[jax.ShapeDtypeStruct((MAX_POS, DP), jnp.int32)] * 6,
)


def kernel(coordinates, width, height, top_left_x, top_left_y,
           bottom_right_x, bottom_right_y, width_emb, height_emb):
    cidx = coordinates.reshape(TOK, 4).astype(jnp.int32)
    c0, c1, c2, c3 = (cidx[:, t] for t in range(4))
    packed = _tc_pack(top_left_x, top_left_y, bottom_right_x, bottom_right_y,
                      width_emb, height_emb)
    out = _spatial_sum(c0, c1, c2, c3,
                       width.astype(jnp.int32), height.astype(jnp.int32),
                       *packed)
    return out.reshape(B, L, D)


# speedup vs baseline: 3.6968x; 1.0004x over previous
import functools

import jax
import jax.numpy as jnp
from jax import lax
from jax.experimental import pallas as pl
from jax.experimental.pallas import tpu as pltpu
from jax.experimental.pallas import tpu_sc as plsc

B, L, D = 4, 4096, 768
MAX_POS = 1024
NC, NS, LANES = 2, 16, 16
NW = NC * NS
TOK = B * L
TPW = TOK // NW
C = 16
NCH = TPW // C
DP = D // 2
DV = DP // LANES


def _sc_body(c0, c1, c2, c3, w, h, tlx, tly, brx, bry, wemb, hemb, out,
             idx_v, stage, obuf, whp, wv, hv,
             isem, gsem0, gsem1, ssem0, ssem1):
    wid = lax.axis_index("s") * NC + lax.axis_index("c")
    base = pl.multiple_of(wid * TPW, TPW)
    b = wid // (NW // B)

    stage_bf = stage.bitcast(jnp.bfloat16)
    whp_bf = whp.bitcast(jnp.bfloat16)

    descs = [pltpu.async_copy(c.at[pl.ds(base, TPW)], idx_v.at[t], isem)
             for t, c in enumerate((c0, c1, c2, c3))]
    bvec = jnp.full((LANES,), b, jnp.int32)
    dw = pltpu.async_copy(w.at[bvec], wv, isem)
    dh = pltpu.async_copy(h.at[bvec], hv, isem)
    for d in descs:
        d.wait()

    @plsc.parallel_loop(0, TPW // LANES, step=1)
    def _clamp(j):
        sl = pl.ds(pl.multiple_of(j * LANES, LANES), LANES)
        for t in range(4):
            idx_v[t, sl] = jnp.minimum(idx_v[t, sl], MAX_POS - 1)

    dw.wait()
    dh.wait()
    wv[...] = jnp.minimum(wv[...], MAX_POS - 1)
    hv[...] = jnp.minimum(hv[...], MAX_POS - 1)
    d1 = pltpu.async_copy(wemb.at[wv], stage.at[0, 0, pl.ds(0, LANES)], isem)
    d2 = pltpu.async_copy(hemb.at[hv], stage.at[0, 1, pl.ds(0, LANES)], isem)
    d1.wait()
    d2.wait()
    pr = pl.ds(0, 2)
    for s in range(DV):
        sl = pl.ds(s * LANES, LANES)
        whp_bf[pr, sl] = stage_bf[0, 0, pr, sl] + stage_bf[0, 1, pr, sl]
    whvecs = [whp_bf[pr, pl.ds(s * LANES, LANES)] for s in range(DV)]

    def gather_descs(k, p, sem):
        ksl = pl.ds(pl.multiple_of(k * C, C), C)
        return [pltpu.make_async_copy(tab.at[idx_v.at[t, ksl]],
                                      stage.at[p, t], sem)
                for t, tab in enumerate((tlx, tly, brx, bry))]

    def issue_gathers(k, p, sem):
        for d in gather_descs(k, p, sem):
            d.start()

    def wait_gathers(k, p, sem):
        for d in gather_descs(k, p, sem):
            d.wait()

    def scatter_desc(k, p, sem):
        koff = pl.multiple_of(k * C, C)
        return pltpu.make_async_copy(obuf.at[p],
                                     out.at[pl.ds(base + koff, C)], sem)

    def process(p):
        @plsc.parallel_loop(0, C, step=1)
        def body(i):
            rs = pl.ds(pl.multiple_of(2 * i, 2), 2)
            for s in range(DV):
                sl = pl.ds(s * LANES, LANES)
                a = stage_bf[p, 0, rs, sl] + stage_bf[p, 1, rs, sl]
                bb = stage_bf[p, 2, rs, sl] + stage_bf[p, 3, rs, sl]
                stage_bf[p, 0, rs, sl] = a + bb + whvecs[s]
            for s in range(DV):
                sl = pl.ds(s * LANES, LANES)
                wrd = stage[p, 0, i, sl]
                obuf[p, i, sl] = lax.bitcast_convert_type(
                    wrd << 16, jnp.float32)
                obuf[p, i, pl.ds(DP + s * LANES, LANES)] = (
                    lax.bitcast_convert_type(wrd & jnp.int32(-65536),
                                             jnp.float32))

    issue_gathers(0, 0, gsem0)

    def pair(j, carry):
        k0 = pl.multiple_of(2 * j, 2)
        k1 = k0 + 1
        wait_gathers(k0, 0, gsem0)
        issue_gathers(k1, 1, gsem1)
        @pl.when(j > 0)
        def _():
            scatter_desc(k0, 0, ssem0).wait()
        process(0)
        scatter_desc(k0, 0, ssem0).start()
        wait_gathers(k1, 1, gsem1)

        @pl.when(k1 + 1 < NCH)
        def _():
            issue_gathers(k1 + 1, 0, gsem0)
        @pl.when(j > 0)
        def _():
            scatter_desc(k1, 1, ssem1).wait()
        process(1)
        scatter_desc(k1, 1, ssem1).start()
        return carry
    lax.fori_loop(0, NCH // 2, pair, 0)
    scatter_desc(NCH - 2, 0, ssem0).wait()
    scatter_desc(NCH - 1, 1, ssem1).wait()


_mesh = plsc.VectorSubcoreMesh(core_axis_name="c", subcore_axis_name="s")

_spatial_sum = functools.partial(
    pl.kernel,
    out_type=jax.ShapeDtypeStruct((TOK, D), jnp.float32),
    mesh=_mesh,
    scratch_types=[
        pltpu.VMEM((4, TPW), jnp.int32),
        pltpu.VMEM((2, 4, C, DP), jnp.int32),
        pltpu.VMEM((2, C, D), jnp.float32),
        pltpu.VMEM((1, DP), jnp.int32),
        pltpu.VMEM((LANES,), jnp.int32),
        pltpu.VMEM((LANES,), jnp.int32),
        pltpu.SemaphoreType.DMA,
        pltpu.SemaphoreType.DMA,
        pltpu.SemaphoreType.DMA,
        pltpu.SemaphoreType.DMA,
        pltpu.SemaphoreType.DMA,
    ],
)(_sc_body)


def _tc_pack_body(*refs):
    for i_ref, o_ref in zip(refs[:6], refs[6:]):
        xb = i_ref[...].astype(jnp.bfloat16)
        u = lax.bitcast_convert_type(xb, jnp.uint16).astype(jnp.int32)
        lo, hi = u[:, :DP], u[:, DP:]
        o_ref[...] = lo | (hi << 16)


_PBR = 256

_tc_pack = pl.pallas_call(
    _tc_pack_body,
    grid=(MAX_POS // _PBR,),
    in_specs=[pl.BlockSpec((_PBR, D), lambda i: (i, 0))] * 6,
    out_specs=[pl.BlockSpec((_PBR, DP), lambda i: (i, 0))] * 6,
    out_shape=[jax.ShapeDtypeStruct((MAX_POS, DP), jnp.int32)] * 6,
)


def kernel(coordinates, width, height, top_left_x, top_left_y,
           bottom_right_x, bottom_right_y, width_emb, height_emb):
    cidx = coordinates.reshape(TOK, 4).astype(jnp.int32)
    c0, c1, c2, c3 = (cidx[:, t] for t in range(4))
    packed = _tc_pack(top_left_x, top_left_y, bottom_right_x, bottom_right_y,
                      width_emb, height_emb)
    out = _spatial_sum(c0, c1, c2, c3,
                       width.astype(jnp.int32), height.astype(jnp.int32),
                       *packed)
    return out.reshape(B, L, D)


# speedup vs baseline: 3.7076x; 1.0029x over previous
import functools

import jax
import jax.numpy as jnp
from jax import lax
from jax.experimental import pallas as pl
from jax.experimental.pallas import tpu as pltpu
from jax.experimental.pallas import tpu_sc as plsc

B, L, D = 4, 4096, 768
MAX_POS = 1024
NC, NS, LANES = 2, 16, 16
NW = NC * NS
TOK = B * L
TPW = TOK // NW
C = 16
NCH = TPW // C
DP = D // 2
DV = DP // LANES


def _sc_body(c0, c1, c2, c3, w, h, tab4, wemb, hemb, out,
             idx_v, idx2, stage, obuf, whp, wv, hv,
             isem, gsem0, gsem1, ssem0, ssem1):
    wid = lax.axis_index("s") * NC + lax.axis_index("c")
    base = pl.multiple_of(wid * TPW, TPW)
    b = wid // (NW // B)

    stage_bf = stage.bitcast(jnp.bfloat16)
    whp_bf = whp.bitcast(jnp.bfloat16)

    descs = [pltpu.async_copy(c.at[pl.ds(base, TPW)], idx_v.at[t], isem)
             for t, c in enumerate((c0, c1, c2, c3))]
    bvec = jnp.full((LANES,), b, jnp.int32)
    dw = pltpu.async_copy(w.at[bvec], wv, isem)
    dh = pltpu.async_copy(h.at[bvec], hv, isem)
    for d in descs:
        d.wait()

    @plsc.parallel_loop(0, NCH, step=1)
    def _clamp(k):
        sl = pl.ds(pl.multiple_of(k * C, C), C)
        for t in range(4):
            v = jnp.minimum(idx_v[t, sl], MAX_POS - 1) + t * MAX_POS
            idx2[k, pl.ds(t * C, C)] = v

    dw.wait()
    dh.wait()
    wv[...] = jnp.minimum(wv[...], MAX_POS - 1)
    hv[...] = jnp.minimum(hv[...], MAX_POS - 1)
    d1 = pltpu.async_copy(wemb.at[wv], stage.at[0, 0, pl.ds(0, LANES)], isem)
    d2 = pltpu.async_copy(hemb.at[hv], stage.at[0, 1, pl.ds(0, LANES)], isem)
    d1.wait()
    d2.wait()
    pr = pl.ds(0, 2)
    for s in range(DV):
        sl = pl.ds(s * LANES, LANES)
        whp_bf[pr, sl] = stage_bf[0, 0, pr, sl] + stage_bf[0, 1, pr, sl]
    whvecs = [whp_bf[pr, pl.ds(s * LANES, LANES)] for s in range(DV)]

    def gather_desc(k, p, sem):
        rows = stage.at[p].reshape(4 * C, DP)
        return pltpu.make_async_copy(tab4.at[idx2.at[k]], rows, sem)

    def issue_gathers(k, p, sem):
        gather_desc(k, p, sem).start()

    def wait_gathers(k, p, sem):
        gather_desc(k, p, sem).wait()

    def scatter_desc(k, p, sem):
        koff = pl.multiple_of(k * C, C)
        return pltpu.make_async_copy(obuf.at[p],
                                     out.at[pl.ds(base + koff, C)], sem)

    def process(p):
        @plsc.parallel_loop(0, C, step=1)
        def body(i):
            rs = pl.ds(pl.multiple_of(2 * i, 2), 2)
            for s in range(DV):
                sl = pl.ds(s * LANES, LANES)
                a = stage_bf[p, 0, rs, sl] + stage_bf[p, 1, rs, sl]
                bb = stage_bf[p, 2, rs, sl] + stage_bf[p, 3, rs, sl]
                stage_bf[p, 0, rs, sl] = a + bb + whvecs[s]
            for s in range(DV):
                sl = pl.ds(s * LANES, LANES)
                wrd = stage[p, 0, i, sl]
                obuf[p, i, sl] = lax.bitcast_convert_type(
                    wrd << 16, jnp.float32)
                obuf[p, i, pl.ds(DP + s * LANES, LANES)] = (
                    lax.bitcast_convert_type(wrd & jnp.int32(-65536),
                                             jnp.float32))

    issue_gathers(0, 0, gsem0)

    def pair(j, carry):
        k0 = pl.multiple_of(2 * j, 2)
        k1 = k0 + 1
        wait_gathers(k0, 0, gsem0)
        issue_gathers(k1, 1, gsem1)
        @pl.when(j > 0)
        def _():
            scatter_desc(k0, 0, ssem0).wait()
        process(0)
        scatter_desc(k0, 0, ssem0).start()
        wait_gathers(k1, 1, gsem1)

        @pl.when(k1 + 1 < NCH)
        def _():
            issue_gathers(k1 + 1, 0, gsem0)
        @pl.when(j > 0)
        def _():
            scatter_desc(k1, 1, ssem1).wait()
        process(1)
        scatter_desc(k1, 1, ssem1).start()
        return carry
    lax.fori_loop(0, NCH // 2, pair, 0)
    scatter_desc(NCH - 2, 0, ssem0).wait()
    scatter_desc(NCH - 1, 1, ssem1).wait()


_mesh = plsc.VectorSubcoreMesh(core_axis_name="c", subcore_axis_name="s")

_spatial_sum = functools.partial(
    pl.kernel,
    out_type=jax.ShapeDtypeStruct((TOK, D), jnp.float32),
    mesh=_mesh,
    scratch_types=[
        pltpu.VMEM((4, TPW), jnp.int32),
        pltpu.VMEM((NCH, 4 * C), jnp.int32),
        pltpu.VMEM((2, 4, C, DP), jnp.int32),
        pltpu.VMEM((2, C, D), jnp.float32),
        pltpu.VMEM((1, DP), jnp.int32),
        pltpu.VMEM((LANES,), jnp.int32),
        pltpu.VMEM((LANES,), jnp.int32),
        pltpu.SemaphoreType.DMA,
        pltpu.SemaphoreType.DMA,
        pltpu.SemaphoreType.DMA,
        pltpu.SemaphoreType.DMA,
        pltpu.SemaphoreType.DMA,
    ],
)(_sc_body)


def _pack_block(x):
    xb = x.astype(jnp.bfloat16)
    u = lax.bitcast_convert_type(xb, jnp.uint16).astype(jnp.int32)
    return u[:, :DP] | (u[:, DP:] << 16)


def _tc_pack_body(*refs):
    ins, (o4, owemb, ohemb) = refs[:6], refs[6:]
    for t in range(4):
        o4[t] = _pack_block(ins[t][...])
    owemb[...] = _pack_block(ins[4][...])
    ohemb[...] = _pack_block(ins[5][...])


_PBR = 256

_tc_pack = pl.pallas_call(
    _tc_pack_body,
    grid=(MAX_POS // _PBR,),
    in_specs=[pl.BlockSpec((_PBR, D), lambda i: (i, 0))] * 6,
    out_specs=[pl.BlockSpec((4, _PBR, DP), lambda i: (0, i, 0)),
               pl.BlockSpec((_PBR, DP), lambda i: (i, 0)),
               pl.BlockSpec((_PBR, DP), lambda i: (i, 0))],
    out_shape=[jax.ShapeDtypeStruct((4, MAX_POS, DP), jnp.int32),
               jax.ShapeDtypeStruct((MAX_POS, DP), jnp.int32),
               jax.ShapeDtypeStruct((MAX_POS, DP), jnp.int32)],
)


def kernel(coordinates, width, height, top_left_x, top_left_y,
           bottom_right_x, bottom_right_y, width_emb, height_emb):
    cidx = coordinates.reshape(TOK, 4).astype(jnp.int32)
    c0, c1, c2, c3 = (cidx[:, t] for t in range(4))
    tab4, wemb_p, hemb_p = _tc_pack(top_left_x, top_left_y,
                                    bottom_right_x, bottom_right_y,
                                    width_emb, height_emb)
    out = _spatial_sum(c0, c1, c2, c3,
                       width.astype(jnp.int32), height.astype(jnp.int32),
                       tab4.reshape(4 * MAX_POS, DP), wemb_p, hemb_p)
    return out.reshape(B, L, D)
